# baseline (speedup 1.0000x reference)
import functools

import jax
import jax.numpy as jnp
from jax import lax
from jax.experimental import pallas as pl
from jax.experimental.pallas import tpu as pltpu
from jax.experimental.pallas import tpu_sc as plsc

_B = 32
_ROW = 160000
_NCH = 10
_CHUNK = _ROW // _NCH
_TOT = 2 * _B * _NCH
_NW = 32
_NROWS = _B * _NCH
_RPW = _NROWS // _NW
_K = 2
_NG = _RPW // _K
_SK = 32
_SNG = (_NROWS // 2) // _SK


def _index_table():
    perm = jnp.argsort(jax.random.uniform(jax.random.key(42), (_B,)))
    idx_noise = (perm[:, None] * _NCH + jnp.arange(_NCH)[None, :]).reshape(-1)
    return idx_noise.astype(jnp.int32).reshape(_NW, _NG, _K)


def _make_remix():
    mesh = plsc.VectorSubcoreMesh(core_axis_name="c", subcore_axis_name="s")

    @functools.partial(
        pl.kernel,
        out_type=jax.ShapeDtypeStruct((_TOT, 1, _CHUNK), jnp.float32),
        mesh=mesh,
        scratch_types=[
            pltpu.VMEM((_NG, _K), jnp.int32),
            pltpu.VMEM((_K, 1, _CHUNK), jnp.float32),
            pltpu.VMEM((_K, 1, _CHUNK), jnp.float32),
            pltpu.VMEM_SHARED((_SK, 1, _CHUNK), jnp.float32),
            pltpu.VMEM_SHARED((_SK, 1, _CHUNK), jnp.float32),
            pltpu.SemaphoreType.DMA,
            pltpu.SemaphoreType.DMA,
            pltpu.SemaphoreType.DMA,
            pltpu.SemaphoreType.DMA,
            pltpu.SemaphoreType.DMA,
            pltpu.SemaphoreType.DMA,
            pltpu.SemaphoreType.DMA,
            pltpu.SemaphoreType.DMA,
        ],
    )
    def remix(src_hbm, idx_hbm, out_hbm, idx_v, nb0, nb1, sb0, sb1,
              ng0, ng1, ns0, ns1, si0, si1, so0, so1):
        s_idx = lax.axis_index("s")
        c_idx = lax.axis_index("c")
        wid = s_idx * 2 + c_idx
        nbase = wid * _RPW
        pltpu.sync_copy(idx_hbm.at[wid], idx_v)

        nbufs, ngsem, nssem = [nb0, nb1], [ng0, ng1], [ns0, ns1]
        sbufs, sisem, sosem = [sb0, sb1], [si0, si1], [so0, so1]
        cbase = _NROWS + c_idx * (_SNG * _SK)

        def in_d(g):
            return pltpu.make_async_copy(
                src_hbm.at[pl.ds(cbase + g * _SK, _SK)],
                sbufs[g % 2], sisem[g % 2])

        def out_d(g):
            return pltpu.make_async_copy(
                sbufs[g % 2],
                out_hbm.at[pl.ds(cbase + g * _SK, _SK)],
                sosem[g % 2])

        is_spw = s_idx == 0

        @pl.when(is_spw)
        def _prime():
            in_d(0).start()

        gh = [None] * _NG
        sh = [None] * _NG
        for g in range(_NG):
            if g >= 2:
                sh[g - 2].wait()
            gh[g] = pltpu.async_copy(
                src_hbm.at[idx_v.at[g]], nbufs[g % 2], ngsem[g % 2]
            )

            @pl.when(is_spw)
            def _spm(g=g):
                in_d(g).wait()
                if g >= 1:
                    out_d(g - 1).wait()
                if g + 1 < _SNG:
                    in_d(g + 1).start()
                out_d(g).start()

            if g >= 1:
                gh[g - 1].wait()
                sh[g - 1] = pltpu.async_copy(
                    nbufs[(g - 1) % 2],
                    out_hbm.at[pl.ds(nbase + (g - 1) * _K, _K)],
                    nssem[(g - 1) % 2],
                )
        gh[_NG - 1].wait()
        sh[_NG - 1] = pltpu.async_copy(
            nbufs[(_NG - 1) % 2],
            out_hbm.at[pl.ds(nbase + (_NG - 1) * _K, _K)],
            nssem[(_NG - 1) % 2],
        )
        sh[_NG - 2].wait()
        sh[_NG - 1].wait()

        @pl.when(is_spw)
        def _drain():
            out_d(_SNG - 1).wait()

    return remix


_remix = _make_remix()


def kernel(sources):
    src = sources.reshape(_TOT, 1, _CHUNK)
    out = _remix(src, _index_table())
    return out.reshape(2, _B, 1, _ROW)
